# baseline (speedup 1.0000x reference)
import functools

import jax
import jax.numpy as jnp
from jax import lax
from jax.experimental import pallas as pl
from jax.experimental.pallas import tpu as pltpu
from jax.experimental.pallas import tpu_sc as plsc

_B, _L, _K, _D = 4096, 200, 4, 128
_NC, _NS = 2, 16
_NW = _NC * _NS
_SC_ROWS = 1536
_ROWS_W = _SC_ROWS // _NW
_NFULL = _L // 16
_TAIL = _L - 16
_NG = _D // 16
_TC_TB = 256

_DNUMS = lax.GatherDimensionNumbers(
    offset_dims=(), collapsed_slice_dims=(0,), start_index_map=(0,))


def _perm(x, idx):
    return lax.gather(x, idx[:, None], _DNUMS, (1,),
                      mode=lax.GatherScatterMode.PROMISE_IN_BOUNDS)


def _sc_body(reads_hbm, table_hbm, out_hbm, buf, tbl, obuf):
    wid = lax.axis_index("s") * _NC + lax.axis_index("c")
    base = wid * _ROWS_W
    pltpu.sync_copy(reads_hbm.at[pl.ds(base, _ROWS_W), :], buf)
    pltpu.sync_copy(table_hbm, tbl)

    lanes = lax.iota(jnp.int32, 16)
    mlow = lanes < 8
    zero = jnp.zeros((16,), jnp.int32)
    inv = jnp.float32(1.0 / _L)
    perm_idx = [lanes ^ sh for sh in (1, 2, 4, 8)]
    lb0 = lanes & 1
    lb1 = (lanes >> 1) & 1
    lut = lb0 + (lb1 << 8) + ((lb0 & lb1) << 16)
    t0v = [tbl[0, pl.ds(d * 16, 16)] for d in range(_NG)]
    dvec = [[(tbl[k, pl.ds(d * 16, 16)] - t0v[d]) * inv
             for d in range(_NG)] for k in range(1, _K)]

    def row_body(r, carry):
        acc = zero
        for k in range(_NFULL):
            acc = acc + _perm(lut, buf[r, pl.ds(k * 16, 16)])
        tail = _perm(lut, buf[r, pl.ds(_TAIL, 16)])
        acc = acc + jnp.where(mlow, zero, tail)
        for pidx in perm_idx:
            acc = acc + _perm(acc, pidx)
        f3 = (acc >> 16).astype(jnp.float32)
        f1 = ((acc & 255).astype(jnp.float32)) - f3
        f2 = (((acc >> 8) & 255).astype(jnp.float32)) - f3
        for d in range(_NG):
            obuf[r, pl.ds(d * 16, 16)] = (
                t0v[d] + f1 * dvec[0][d] + f2 * dvec[1][d] + f3 * dvec[2][d])
        return carry

    lax.fori_loop(0, _ROWS_W, row_body, 0)
    pltpu.sync_copy(obuf, out_hbm.at[pl.ds(base, _ROWS_W), :])


def _tc_block(reads_ref, table_ref, out_ref, *, inv_len):
    r = reads_ref[...]
    b0 = (r & 1).astype(jnp.float32)
    b1 = (r >> 1).astype(jnp.float32)
    s0 = jnp.sum(b0, axis=1, keepdims=True)
    s1 = jnp.sum(b1, axis=1, keepdims=True)
    c3 = jnp.sum(b0 * b1, axis=1, keepdims=True)
    c1 = s0 - c3
    c2 = s1 - c3
    c0 = jnp.float32(r.shape[1]) - c1 - c2 - c3
    t = table_ref[...]
    acc = (c0 * t[0, :][None, :] + c1 * t[1, :][None, :]
           + c2 * t[2, :][None, :] + c3 * t[3, :][None, :])
    out_ref[...] = acc * inv_len


def kernel(reads, table):
    reads = reads.astype(jnp.int32)

    sc_run = functools.partial(
        pl.kernel,
        out_type=jax.ShapeDtypeStruct((_SC_ROWS, _D), jnp.float32),
        mesh=plsc.VectorSubcoreMesh(core_axis_name="c", subcore_axis_name="s"),
        scratch_types=[
            pltpu.VMEM((_ROWS_W, _L), jnp.int32),
            pltpu.VMEM((_K, _D), jnp.float32),
            pltpu.VMEM((_ROWS_W, _D), jnp.float32),
        ],
        compiler_params=pltpu.CompilerParams(use_tc_tiling_on_sc=True),
    )(_sc_body)
    out_sc = sc_run(reads[:_SC_ROWS], table)

    tc_rows = _B - _SC_ROWS
    skip = _SC_ROWS // _TC_TB
    out_tc = pl.pallas_call(
        functools.partial(_tc_block, inv_len=1.0 / _L),
        grid=(tc_rows // _TC_TB,),
        in_specs=[
            pl.BlockSpec((_TC_TB, _L), lambda i: (i + skip, 0)),
            pl.BlockSpec((_K, _D), lambda i: (0, 0)),
        ],
        out_specs=pl.BlockSpec((_TC_TB, _D), lambda i: (i, 0)),
        out_shape=jax.ShapeDtypeStruct((tc_rows, _D), jnp.float32),
    )(reads, table)

    return jnp.concatenate([out_sc, out_tc], axis=0)


# speedup vs baseline: 1.0903x; 1.0903x over previous
import functools

import jax
import jax.numpy as jnp
from jax import lax
from jax.experimental import pallas as pl
from jax.experimental.pallas import tpu as pltpu
from jax.experimental.pallas import tpu_sc as plsc

_B, _L, _K, _D = 4096, 200, 4, 128
_NC, _NS = 2, 16
_NW = _NC * _NS
_SC_ROWS = 2048
_ROWS_W = _SC_ROWS // _NW
_NFULL = _L // 16
_TAIL = _L - 16
_NG = _D // 16
_TC_TB = 512

_DNUMS = lax.GatherDimensionNumbers(
    offset_dims=(), collapsed_slice_dims=(0,), start_index_map=(0,))


def _perm(x, idx):
    return lax.gather(x, idx[:, None], _DNUMS, (1,),
                      mode=lax.GatherScatterMode.PROMISE_IN_BOUNDS)


def _sc_body(reads_hbm, table_hbm, out_hbm, buf, tbl, obuf):
    wid = lax.axis_index("s") * _NC + lax.axis_index("c")
    base = wid * _ROWS_W
    pltpu.sync_copy(reads_hbm.at[pl.ds(base, _ROWS_W), :], buf)
    pltpu.sync_copy(table_hbm, tbl)

    lanes = lax.iota(jnp.int32, 16)
    mlow = lanes < 8
    zero = jnp.zeros((16,), jnp.int32)
    inv = jnp.float32(1.0 / _L)
    perm_idx = [lanes ^ sh for sh in (1, 2, 4, 8)]
    lb0 = lanes & 1
    lb1 = (lanes >> 1) & 1
    lut = lb0 + (lb1 << 8) + ((lb0 & lb1) << 16)
    t0v = [tbl[0, pl.ds(d * 16, 16)] for d in range(_NG)]
    dvec = [[(tbl[k, pl.ds(d * 16, 16)] - t0v[d]) * inv
             for d in range(_NG)] for k in range(1, _K)]

    def row_body(r, carry):
        acc = zero
        for k in range(_NFULL):
            acc = acc + _perm(lut, buf[r, pl.ds(k * 16, 16)])
        tail = _perm(lut, buf[r, pl.ds(_TAIL, 16)])
        acc = acc + jnp.where(mlow, zero, tail)
        for pidx in perm_idx:
            acc = acc + _perm(acc, pidx)
        f3 = (acc >> 16).astype(jnp.float32)
        f1 = ((acc & 255).astype(jnp.float32)) - f3
        f2 = (((acc >> 8) & 255).astype(jnp.float32)) - f3
        for d in range(_NG):
            obuf[r, pl.ds(d * 16, 16)] = (
                t0v[d] + f1 * dvec[0][d] + f2 * dvec[1][d] + f3 * dvec[2][d])
        return carry

    lax.fori_loop(0, _ROWS_W, row_body, 0)
    pltpu.sync_copy(obuf, out_hbm.at[pl.ds(base, _ROWS_W), :])


def _tc_block(reads_ref, table_ref, out_ref, *, inv_len):
    r = reads_ref[...]
    b0 = (r & 1).astype(jnp.float32)
    b1 = (r >> 1).astype(jnp.float32)
    s0 = jnp.sum(b0, axis=1, keepdims=True)
    s1 = jnp.sum(b1, axis=1, keepdims=True)
    c3 = jnp.sum(b0 * b1, axis=1, keepdims=True)
    c1 = s0 - c3
    c2 = s1 - c3
    c0 = jnp.float32(r.shape[1]) - c1 - c2 - c3
    t = table_ref[...]
    acc = (c0 * t[0, :][None, :] + c1 * t[1, :][None, :]
           + c2 * t[2, :][None, :] + c3 * t[3, :][None, :])
    out_ref[...] = acc * inv_len


def kernel(reads, table):
    reads = reads.astype(jnp.int32)

    sc_run = functools.partial(
        pl.kernel,
        out_type=jax.ShapeDtypeStruct((_SC_ROWS, _D), jnp.float32),
        mesh=plsc.VectorSubcoreMesh(core_axis_name="c", subcore_axis_name="s"),
        scratch_types=[
            pltpu.VMEM((_ROWS_W, _L), jnp.int32),
            pltpu.VMEM((_K, _D), jnp.float32),
            pltpu.VMEM((_ROWS_W, _D), jnp.float32),
        ],
        compiler_params=pltpu.CompilerParams(use_tc_tiling_on_sc=True),
    )(_sc_body)
    out_sc = sc_run(reads[:_SC_ROWS], table)

    tc_rows = _B - _SC_ROWS
    out_tc = pl.pallas_call(
        functools.partial(_tc_block, inv_len=1.0 / _L),
        grid=(tc_rows // _TC_TB,),
        in_specs=[
            pl.BlockSpec((_TC_TB, _L), lambda i: (i, 0)),
            pl.BlockSpec((_K, _D), lambda i: (0, 0)),
        ],
        out_specs=pl.BlockSpec((_TC_TB, _D), lambda i: (i, 0)),
        out_shape=jax.ShapeDtypeStruct((tc_rows, _D), jnp.float32),
    )(reads[_SC_ROWS:], table)

    return jnp.concatenate([out_sc, out_tc], axis=0)


# speedup vs baseline: 1.1095x; 1.0176x over previous
import functools

import jax
import jax.numpy as jnp
from jax import lax
from jax.experimental import pallas as pl
from jax.experimental.pallas import tpu as pltpu
from jax.experimental.pallas import tpu_sc as plsc

_B, _L, _K, _D = 4096, 200, 4, 128
_NC, _NS = 2, 16
_NW = _NC * _NS
_SC_ROWS = 2048
_ROWS_W = _SC_ROWS // _NW
_NFULL = _L // 16
_TAIL = _L - 16
_NG = _D // 16
_TC_TB = 512

_DNUMS = lax.GatherDimensionNumbers(
    offset_dims=(), collapsed_slice_dims=(0,), start_index_map=(0,))


def _perm(x, idx):
    return lax.gather(x, idx[:, None], _DNUMS, (1,),
                      mode=lax.GatherScatterMode.PROMISE_IN_BOUNDS)


def _sc_body(reads_hbm, table_hbm, out_hbm, buf, tbl, obuf):
    wid = lax.axis_index("s") * _NC + lax.axis_index("c")
    base = wid * _ROWS_W
    pltpu.sync_copy(reads_hbm.at[pl.ds(base, _ROWS_W), :], buf)
    pltpu.sync_copy(table_hbm, tbl)

    lanes = lax.iota(jnp.int32, 16)
    mlow = lanes < 8
    zero = jnp.zeros((16,), jnp.int32)
    inv = jnp.float32(1.0 / _L)
    perm_idx = [lanes ^ sh for sh in (1, 2, 4, 8)]
    lb0 = lanes & 1
    lb1 = (lanes >> 1) & 1
    lut = lb0 + (lb1 << 8) + ((lb0 & lb1) << 16)
    t0v = [tbl[0, pl.ds(d * 16, 16)] for d in range(_NG)]
    dvec = [[(tbl[k, pl.ds(d * 16, 16)] - t0v[d]) * inv
             for d in range(_NG)] for k in range(1, _K)]

    def histogram(r):
        acc0 = zero
        acc1 = zero
        for k in range(0, _NFULL, 2):
            acc0 = acc0 + _perm(lut, buf[r, pl.ds(k * 16, 16)])
            acc1 = acc1 + _perm(lut, buf[r, pl.ds((k + 1) * 16, 16)])
        tail = _perm(lut, buf[r, pl.ds(_TAIL, 16)])
        acc = acc0 + acc1 + jnp.where(mlow, zero, tail)
        for pidx in perm_idx:
            acc = acc + _perm(acc, pidx)
        return acc

    def emit(r, acc):
        f3 = (acc >> 16).astype(jnp.float32)
        f1 = ((acc & 255).astype(jnp.float32)) - f3
        f2 = (((acc >> 8) & 255).astype(jnp.float32)) - f3
        for d in range(_NG):
            obuf[r, pl.ds(d * 16, 16)] = (
                t0v[d] + f1 * dvec[0][d] + f2 * dvec[1][d] + f3 * dvec[2][d])

    def pair_body(p, carry):
        r = 2 * p
        acc_a = histogram(r)
        acc_b = histogram(r + 1)
        emit(r, acc_a)
        emit(r + 1, acc_b)
        return carry

    lax.fori_loop(0, _ROWS_W // 2, pair_body, 0)
    pltpu.sync_copy(obuf, out_hbm.at[pl.ds(base, _ROWS_W), :])


def _tc_block(reads_ref, table_ref, out_ref, *, inv_len):
    r = reads_ref[...]
    b0 = (r & 1).astype(jnp.float32)
    b1 = (r >> 1).astype(jnp.float32)
    s0 = jnp.sum(b0, axis=1, keepdims=True)
    s1 = jnp.sum(b1, axis=1, keepdims=True)
    c3 = jnp.sum(b0 * b1, axis=1, keepdims=True)
    c1 = s0 - c3
    c2 = s1 - c3
    c0 = jnp.float32(r.shape[1]) - c1 - c2 - c3
    t = table_ref[...]
    acc = (c0 * t[0, :][None, :] + c1 * t[1, :][None, :]
           + c2 * t[2, :][None, :] + c3 * t[3, :][None, :])
    out_ref[...] = acc * inv_len


def kernel(reads, table):
    reads = reads.astype(jnp.int32)

    sc_run = functools.partial(
        pl.kernel,
        out_type=jax.ShapeDtypeStruct((_SC_ROWS, _D), jnp.float32),
        mesh=plsc.VectorSubcoreMesh(core_axis_name="c", subcore_axis_name="s"),
        scratch_types=[
            pltpu.VMEM((_ROWS_W, _L), jnp.int32),
            pltpu.VMEM((_K, _D), jnp.float32),
            pltpu.VMEM((_ROWS_W, _D), jnp.float32),
        ],
        compiler_params=pltpu.CompilerParams(use_tc_tiling_on_sc=True),
    )(_sc_body)
    out_sc = sc_run(reads[:_SC_ROWS], table)

    tc_rows = _B - _SC_ROWS
    out_tc = pl.pallas_call(
        functools.partial(_tc_block, inv_len=1.0 / _L),
        grid=(tc_rows // _TC_TB,),
        in_specs=[
            pl.BlockSpec((_TC_TB, _L), lambda i: (i, 0)),
            pl.BlockSpec((_K, _D), lambda i: (0, 0)),
        ],
        out_specs=pl.BlockSpec((_TC_TB, _D), lambda i: (i, 0)),
        out_shape=jax.ShapeDtypeStruct((tc_rows, _D), jnp.float32),
    )(reads[_SC_ROWS:], table)

    return jnp.concatenate([out_sc, out_tc], axis=0)
